# baseline (speedup 1.0000x reference)
import jax
import jax.numpy as jnp
from jax.experimental import pallas as pl

N = 100000
C = 80
BETA = 2.0
LOSS_WEIGHT = 1.0

BLK = 2000
NBLK = N // BLK


def _softplus(x):
    return jnp.maximum(x, 0.0) + jnp.log1p(jnp.exp(-jnp.abs(x)))


def _body(pred_ref, label_ref, score_ref, out_ref):
    x = pred_ref[...]
    lab = label_ref[...]
    sc = score_ref[...]

    sig = jax.nn.sigmoid(x)
    sp = _softplus(x)
    neg = sp * (sig * sig) * 0.75

    labc = jnp.clip(lab, 0, C - 1)
    cols = jax.lax.broadcasted_iota(jnp.int32, (BLK, C), 1)
    onehot = cols == labc
    pred_pos = jnp.sum(jnp.where(onehot, x, 0.0), axis=1, keepdims=True)
    neg_at = jnp.sum(jnp.where(onehot, neg, 0.0), axis=1, keepdims=True)

    pos_mask = (lab >= 0) & (lab < C)
    w = jnp.abs(sc - pred_pos)
    pos_loss = (_softplus(pred_pos) - pred_pos * sc) * (w * w)
    corr = jnp.where(pos_mask, pos_loss - neg_at, 0.0)

    bsum = (jnp.sum(neg) + jnp.sum(corr)).reshape(1, 1)

    @pl.when(pl.program_id(0) == 0)
    def _init():
        out_ref[...] = jnp.zeros((1, 1), jnp.float32)

    out_ref[...] += bsum


def kernel(pred, label, score):
    out = pl.pallas_call(
        _body,
        grid=(NBLK,),
        in_specs=[
            pl.BlockSpec((BLK, C), lambda i: (i, 0)),
            pl.BlockSpec((BLK, 1), lambda i: (i, 0)),
            pl.BlockSpec((BLK, 1), lambda i: (i, 0)),
        ],
        out_specs=pl.BlockSpec((1, 1), lambda i: (0, 0)),
        out_shape=jax.ShapeDtypeStruct((1, 1), jnp.float32),
    )(pred, label[:, None], score[:, None])
    return (out[0, 0] * (LOSS_WEIGHT / (N * C))).astype(jnp.float32)
